# Initial kernel scaffold; baseline (speedup 1.0000x reference)
import jax
import jax.numpy as jnp
from jax.experimental import pallas as pl


def kernel(x, classes, memory):
    raise NotImplementedError("write your pallas kernel here")


# speedup vs baseline: 7.6540x; 7.6540x over previous
import functools

import jax
import jax.numpy as jnp
from jax import lax
from jax.experimental import pallas as pl
from jax.experimental.pallas import tpu as pltpu, tpu_sc as plsc

NUM_CLASSES = 1000
CAP = 32
DIM = 1024
BATCH = 4096

SLOTS = NUM_CLASSES * CAP
NUM_TILES = 32
SPT = SLOTS // NUM_TILES
K = 40
NCH = SPT // K
L = 16


def _sc_update(mem_flat, x, mem_idx, src, dst, offs):
    mesh = plsc.VectorSubcoreMesh(core_axis_name="c", subcore_axis_name="s")

    @functools.partial(
        pl.kernel,
        out_type=jax.ShapeDtypeStruct((SLOTS, DIM), jnp.float32),
        mesh=mesh,
        compiler_params=pltpu.CompilerParams(needs_layout_passes=False),
        scratch_types=[
            pltpu.VMEM((K,), jnp.int32),
            pltpu.VMEM((K, DIM), jnp.float32),
            pltpu.VMEM((L, DIM), jnp.float32),
            pltpu.VMEM((BATCH // L, L), jnp.int32),
            pltpu.VMEM((BATCH // L, L), jnp.int32),
            pltpu.VMEM((40,), jnp.int32),
            pltpu.SemaphoreType.DMA,
            pltpu.SemaphoreType.DMA,
        ],
    )
    def k(mem_hbm, x_hbm, mem_idx_hbm, src_hbm, dst_hbm, off_hbm, out_hbm,
          idx_chunk, rows_v, xbuf, src_v, dst_v, off_v, sem, sem2):
        g = lax.axis_index("c") * 16 + lax.axis_index("s")
        base = g * SPT

        pltpu.sync_copy(src_hbm, src_v)
        pltpu.sync_copy(dst_hbm, dst_v)
        pltpu.sync_copy(off_hbm, off_v)

        def chunk_body(ch, carry):
            off = base + ch * K
            pltpu.sync_copy(mem_idx_hbm.at[pl.ds(off, K)], idx_chunk)
            pltpu.async_copy(mem_hbm.at[idx_chunk], rows_v, sem).wait()
            pltpu.sync_copy(rows_v, out_hbm.at[pl.ds(off, K)])
            return carry

        lax.fori_loop(0, NCH, chunk_body, 0)

        gv = jnp.full((L,), g, jnp.int32)
        lo = jnp.max(plsc.load_gather(off_v, [gv]))
        hi = jnp.max(plsc.load_gather(off_v, [gv + 1]))
        t0 = lo // L
        t1 = (hi + (L - 1)) // L

        def sub_body(t, carry):
            pltpu.async_copy(x_hbm.at[src_v.at[t]], xbuf, sem2).wait()
            pltpu.async_copy(xbuf, out_hbm.at[dst_v.at[t]], sem2).wait()
            return carry

        lax.fori_loop(t0, t1, sub_body, 0)

    return k(mem_flat, x, mem_idx, src, dst, offs)


def kernel(x, classes, memory):
    B, C = BATCH, NUM_CLASSES
    pos = jnp.arange(B, dtype=jnp.int32)

    order = jnp.argsort(classes, stable=True).astype(jnp.int32)
    sc = classes[order]
    is_start = jnp.concatenate(
        [jnp.ones((1,), jnp.bool_), sc[1:] != sc[:-1]])
    seg_start = lax.cummax(jnp.where(is_start, pos, 0))
    rank = jnp.zeros((B,), jnp.int32).at[order].set(pos - seg_start)
    counts = jnp.zeros((C,), jnp.int32).at[classes].add(1)

    j = jnp.arange(CAP, dtype=jnp.int32)
    mem_idx = (
        jnp.arange(C, dtype=jnp.int32)[:, None] * CAP
        + jnp.clip(j[None, :] - counts[:, None], 0, CAP - 1)
    ).reshape(-1)

    valid = rank < CAP
    dst = jnp.where(valid, classes * CAP + rank, classes[0] * CAP)
    src = jnp.where(valid, pos, 0)
    o2 = jnp.argsort(dst).astype(jnp.int32)
    dst_s = dst[o2]
    src_s = src[o2]
    ends = jnp.searchsorted(
        dst_s, (jnp.arange(NUM_TILES, dtype=jnp.int32) + 1) * SPT
    ).astype(jnp.int32)
    offs = jnp.concatenate(
        [jnp.zeros((1,), jnp.int32), ends, jnp.zeros((7,), jnp.int32)])

    out = _sc_update(memory.reshape(SLOTS, DIM), x, mem_idx,
                     src_s.reshape(BATCH // L, L), dst_s.reshape(BATCH // L, L),
                     offs)
    return out.reshape(NUM_CLASSES, CAP, DIM)
